# Initial kernel scaffold; baseline (speedup 1.0000x reference)
import jax
import jax.numpy as jnp
from jax.experimental import pallas as pl


def kernel(x, edge_index, edge_attr, W_l, b_l, W_r, b_r, W_e, att, bias):
    raise NotImplementedError("write your pallas kernel here")


# speedup vs baseline: 55.4260x; 55.4260x over previous
import functools

import jax
import jax.numpy as jnp
from jax import lax
from jax.experimental import pallas as pl
from jax.experimental.pallas import tpu as pltpu
from jax.experimental.pallas import tpu_sc as plsc

NC = 2
NS = 16
NW = NC * NS
L = 16

CHUNK = 2048
CROWS = CHUNK // 128


def _cdiv(a, b):
    return (a + b - 1) // b


def _sc_pass1(x_p, src, dst1, dst2, ea2, zeros_na, *, np_, ept):
    n_chunks = ept // CHUNK
    erows_pt = ept // 128
    mesh = plsc.VectorSubcoreMesh(core_axis_name="c", subcore_axis_name="s")

    @functools.partial(
        pl.kernel,
        out_type=[
            jax.ShapeDtypeStruct((NW * ept,), jnp.float32),
            jax.ShapeDtypeStruct((NW * ept,), jnp.float32),
            jax.ShapeDtypeStruct((NC, np_), jnp.float32),
            jax.ShapeDtypeStruct((NC, np_), jnp.float32),
        ],
        mesh=mesh,
        compiler_params=pltpu.CompilerParams(needs_layout_passes=False),
        scratch_types=[
            pltpu.VMEM((np_,), jnp.float32),
            pltpu.VMEM((CHUNK,), jnp.int32),
            pltpu.VMEM((CHUNK,), jnp.int32),
            pltpu.VMEM((CROWS, 128), jnp.int32),
            pltpu.VMEM((CROWS, 128), jnp.float32),
            pltpu.VMEM((CHUNK,), jnp.float32),
            pltpu.VMEM((CHUNK,), jnp.float32),
            pltpu.VMEM((128,), jnp.float32),
            pltpu.VMEM_SHARED((np_,), jnp.float32),
            pltpu.VMEM_SHARED((np_,), jnp.float32),
        ],
    )
    def k(x_hbm, src_hbm, dst1_hbm, dst2_hbm, ea2_hbm, z_hbm,
          s_hbm, d_hbm, deg_hbm, attr_hbm,
          x_v, srcb, dstb1, dstb2, eab, sb, db, ones_v, deg_sh, attr_sh):
        cid = lax.axis_index("c")
        sid = lax.axis_index("s")
        wid = cid * NS + sid

        @pl.when(sid == 0)
        def _():
            pltpu.sync_copy(z_hbm, deg_sh)
            pltpu.sync_copy(z_hbm, attr_sh)

        pltpu.sync_copy(x_hbm, x_v)
        for i in range(8):
            ones_v[pl.ds(i * L, L)] = jnp.ones((L,), jnp.float32)
        plsc.subcore_barrier()

        base = wid * ept
        base_row = wid * erows_pt

        def chunk_body(kk, _):
            off = base + kk * CHUNK
            row = base_row + kk * CROWS
            pltpu.sync_copy(src_hbm.at[pl.ds(off, CHUNK)], srcb)
            pltpu.sync_copy(dst1_hbm.at[pl.ds(off, CHUNK)], dstb1)
            pltpu.sync_copy(dst2_hbm.at[pl.ds(row, CROWS)], dstb2)
            pltpu.sync_copy(ea2_hbm.at[pl.ds(row, CROWS)], eab)

            def gat(i, _):
                sidx = srcb[pl.ds(i * L, L)]
                sb[pl.ds(i * L, L)] = plsc.load_gather(x_v, [sidx])
                didx = dstb1[pl.ds(i * L, L)]
                db[pl.ds(i * L, L)] = plsc.load_gather(x_v, [didx])
                return 0

            lax.fori_loop(0, CHUNK // L, gat, 0)
            pltpu.sync_copy(sb, s_hbm.at[pl.ds(off, CHUNK)])
            pltpu.sync_copy(db, d_hbm.at[pl.ds(off, CHUNK)])

            def sca(j, _):
                pltpu.sync_copy(ones_v, deg_sh.at[dstb2.at[j]], add=True)
                pltpu.sync_copy(eab.at[j], attr_sh.at[dstb2.at[j]], add=True)
                return 0

            lax.fori_loop(0, CROWS, sca, 0)
            return 0

        lax.fori_loop(0, n_chunks, chunk_body, 0)
        plsc.subcore_barrier()

        @pl.when(sid == 0)
        def _():
            pltpu.sync_copy(deg_sh, deg_hbm.at[cid])
            pltpu.sync_copy(attr_sh, attr_hbm.at[cid])

    return k(x_p, src, dst1, dst2, ea2, zeros_na)


def _sc_pass2(dst2, p2, w2, zeros_na, *, np_, ept):
    n_chunks = ept // CHUNK
    erows_pt = ept // 128
    mesh = plsc.VectorSubcoreMesh(core_axis_name="c", subcore_axis_name="s")

    @functools.partial(
        pl.kernel,
        out_type=[
            jax.ShapeDtypeStruct((NC, np_), jnp.float32),
            jax.ShapeDtypeStruct((NC, np_), jnp.float32),
        ],
        mesh=mesh,
        compiler_params=pltpu.CompilerParams(needs_layout_passes=False),
        scratch_types=[
            pltpu.VMEM((CROWS, 128), jnp.int32),
            pltpu.VMEM((CROWS, 128), jnp.float32),
            pltpu.VMEM((CROWS, 128), jnp.float32),
            pltpu.VMEM_SHARED((np_,), jnp.float32),
            pltpu.VMEM_SHARED((np_,), jnp.float32),
        ],
    )
    def k(dst2_hbm, p2_hbm, w2_hbm, z_hbm, pp_hbm, ss_hbm,
          dstb, pb, wb, p_sh, s_sh):
        cid = lax.axis_index("c")
        sid = lax.axis_index("s")
        wid = cid * NS + sid

        @pl.when(sid == 0)
        def _():
            pltpu.sync_copy(z_hbm, p_sh)
            pltpu.sync_copy(z_hbm, s_sh)

        plsc.subcore_barrier()
        base_row = wid * erows_pt

        def chunk_body(kk, _):
            row = base_row + kk * CROWS
            pltpu.sync_copy(dst2_hbm.at[pl.ds(row, CROWS)], dstb)
            pltpu.sync_copy(p2_hbm.at[pl.ds(row, CROWS)], pb)
            pltpu.sync_copy(w2_hbm.at[pl.ds(row, CROWS)], wb)

            def sca(j, _):
                pltpu.sync_copy(pb.at[j], p_sh.at[dstb.at[j]], add=True)
                pltpu.sync_copy(wb.at[j], s_sh.at[dstb.at[j]], add=True)
                return 0

            lax.fori_loop(0, CROWS, sca, 0)
            return 0

        lax.fori_loop(0, n_chunks, chunk_body, 0)
        plsc.subcore_barrier()

        @pl.when(sid == 0)
        def _():
            pltpu.sync_copy(p_sh, pp_hbm.at[cid])
            pltpu.sync_copy(s_sh, ss_hbm.at[cid])

    return k(dst2, p2, w2, zeros_na)


def _raw_block(s, d, a, wl, wr, we, at, bs):
    acc = jnp.zeros_like(s)
    for c in range(32):
        m = s * wl[0, c] + d * wr[0, c] + a * we[0, c] + bs[0, c]
        m = jnp.where(m >= 0.0, m, 0.2 * m)
        acc = acc + m * at[0, c]
    return acc


def _alpha_edges_kernel(wl, wr, we, at, bs, s, d, a, raw, bmax):
    acc = _raw_block(s[...], d[...], a[...], wl, wr, we, at, bs)
    raw[...] = acc
    prev = jnp.where(pl.program_id(0) == 0, -jnp.inf, bmax[0, 0])
    bmax[0, 0] = jnp.maximum(prev, jnp.max(acc))


def _alpha_self_kernel(wl, wr, we, at, bs, x, degp, attrp, raw, bmax):
    deg = degp[0] + degp[1]
    asum = attrp[0] + attrp[1]
    la = asum / jnp.maximum(deg, 1.0)
    xv = x[...]
    acc = _raw_block(xv, xv, la, wl, wr, we, at, bs)
    raw[...] = acc
    prev = jnp.where(pl.program_id(0) == 0, -jnp.inf, bmax[0, 0])
    bmax[0, 0] = jnp.maximum(prev, jnp.max(acc))


def _exp_kernel(g, raw, s, p, w):
    pv = jnp.exp(raw[...] - g[0, 0])
    p[...] = pv
    w[...] = pv * s[...]


def _final_kernel(g, wl, bl, bias, xc, pp, ss, raws, out):
    p_tot = pp[0] + pp[1]
    s_tot = ss[0] + ss[1]
    ps = jnp.exp(raws[...] - g[0, 0])
    p_tot = p_tot + ps
    s_tot = s_tot + ps * xc[...]
    denom = p_tot + 1e-16
    out[...] = (wl[...] * s_tot + bl[...] * p_tot) / denom + bias[...]


def kernel(x, edge_index, edge_attr, W_l, b_l, W_r, b_r, W_e, att, bias):
    n = x.shape[0]
    e = edge_index.shape[1]
    np_ = _cdiv(n, 1024) * 1024
    ept = _cdiv(e, NW * CHUNK) * CHUNK
    e_pad = NW * ept
    erows = e_pad // 128
    nrows = np_ // 128

    xf = x[:, 0]
    x_p = jnp.pad(xf, (0, np_ - n))
    pad = e_pad - e
    src = jnp.concatenate([edge_index[0], jnp.zeros((pad,), jnp.int32)])
    pad_dst = (n + (jnp.arange(pad, dtype=jnp.int32) % 256)).astype(jnp.int32)
    dst = jnp.concatenate([edge_index[1], pad_dst])
    ea = jnp.concatenate([edge_attr[:, 0], jnp.zeros((pad,), jnp.float32)])
    dst2 = dst.reshape(erows, 128)
    ea2 = ea.reshape(erows, 128)
    zeros_na = jnp.zeros((np_,), jnp.float32)

    s_arr, d_arr, degp, attrp = _sc_pass1(
        x_p, src, dst, dst2, ea2, zeros_na, np_=np_, ept=ept)
    s2 = s_arr.reshape(erows, 128)
    d2 = d_arr.reshape(erows, 128)

    wl = W_l.reshape(1, 32)
    wr = W_r.reshape(1, 32)
    we = W_e.reshape(1, 32)
    at2 = att.reshape(1, 32)
    bs = (b_l + b_r).reshape(1, 32)
    bl2 = b_l.reshape(1, 32)
    bias2 = bias.reshape(1, 32)

    smem = pl.BlockSpec(memory_space=pltpu.SMEM)
    be = 128
    grid_e = erows // be
    raw2, bmax_e = pl.pallas_call(
        _alpha_edges_kernel,
        grid=(grid_e,),
        in_specs=[smem] * 5 + [
            pl.BlockSpec((be, 128), lambda i: (i, 0)),
            pl.BlockSpec((be, 128), lambda i: (i, 0)),
            pl.BlockSpec((be, 128), lambda i: (i, 0)),
        ],
        out_specs=[
            pl.BlockSpec((be, 128), lambda i: (i, 0)),
            smem,
        ],
        out_shape=[
            jax.ShapeDtypeStruct((erows, 128), jnp.float32),
            jax.ShapeDtypeStruct((1, 1), jnp.float32),
        ],
    )(wl, wr, we, at2, bs, s2, d2, ea2)

    x3 = x_p.reshape(nrows, 128)
    degp3 = degp.reshape(NC, nrows, 128)
    attrp3 = attrp.reshape(NC, nrows, 128)
    bn = nrows // 7 if nrows % 7 == 0 else nrows
    grid_s = nrows // bn
    raw_self, bmax_s = pl.pallas_call(
        _alpha_self_kernel,
        grid=(grid_s,),
        in_specs=[smem] * 5 + [
            pl.BlockSpec((bn, 128), lambda i: (i, 0)),
            pl.BlockSpec((NC, bn, 128), lambda i: (0, i, 0)),
            pl.BlockSpec((NC, bn, 128), lambda i: (0, i, 0)),
        ],
        out_specs=[
            pl.BlockSpec((bn, 128), lambda i: (i, 0)),
            smem,
        ],
        out_shape=[
            jax.ShapeDtypeStruct((nrows, 128), jnp.float32),
            jax.ShapeDtypeStruct((1, 1), jnp.float32),
        ],
    )(wl, wr, we, at2, bs, x3, degp3, attrp3)

    gmax = jnp.maximum(bmax_e[0, 0], bmax_s[0, 0]).reshape(1, 1)

    p2, w2 = pl.pallas_call(
        _exp_kernel,
        grid=(grid_e,),
        in_specs=[smem] + [
            pl.BlockSpec((be, 128), lambda i: (i, 0)),
            pl.BlockSpec((be, 128), lambda i: (i, 0)),
        ],
        out_specs=[
            pl.BlockSpec((be, 128), lambda i: (i, 0)),
            pl.BlockSpec((be, 128), lambda i: (i, 0)),
        ],
        out_shape=[
            jax.ShapeDtypeStruct((erows, 128), jnp.float32),
            jax.ShapeDtypeStruct((erows, 128), jnp.float32),
        ],
    )(gmax, raw2, s2)

    ppart, spart = _sc_pass2(dst2, p2, w2, zeros_na, np_=np_, ept=ept)

    bf = 256
    grid_f = np_ // bf
    out = pl.pallas_call(
        _final_kernel,
        grid=(grid_f,),
        in_specs=[
            smem,
            pl.BlockSpec((1, 32), lambda i: (0, 0)),
            pl.BlockSpec((1, 32), lambda i: (0, 0)),
            pl.BlockSpec((1, 32), lambda i: (0, 0)),
            pl.BlockSpec((bf, 1), lambda i: (i, 0)),
            pl.BlockSpec((NC, bf, 1), lambda i: (0, i, 0)),
            pl.BlockSpec((NC, bf, 1), lambda i: (0, i, 0)),
            pl.BlockSpec((bf, 1), lambda i: (i, 0)),
        ],
        out_specs=pl.BlockSpec((bf, 32), lambda i: (i, 0)),
        out_shape=jax.ShapeDtypeStruct((np_, 32), jnp.float32),
    )(gmax, wl, bl2, bias2,
      x_p.reshape(np_, 1),
      ppart.reshape(NC, np_, 1),
      spart.reshape(NC, np_, 1),
      raw_self.reshape(np_, 1))

    return out[:n].reshape(1, n * 32)


# speedup vs baseline: 95.0838x; 1.7155x over previous
import functools

import jax
import jax.numpy as jnp
from jax import lax
from jax.experimental import pallas as pl
from jax.experimental.pallas import tpu as pltpu
from jax.experimental.pallas import tpu_sc as plsc

NC = 2
NS = 16
NW = NC * NS

CHUNK = 2048
CROWS = CHUNK // 128


def _cdiv(a, b):
    return (a + b - 1) // b


def _sc_pass1(x_p, src1, dst2, dst1, ea2, zeros_na, *, np_, ept):
    n_chunks = ept // CHUNK
    erows_pt = ept // 128
    mesh = plsc.VectorSubcoreMesh(core_axis_name="c", subcore_axis_name="s")

    @functools.partial(
        pl.kernel,
        out_type=[
            jax.ShapeDtypeStruct((NW * ept,), jnp.float32),
            jax.ShapeDtypeStruct((NW * ept,), jnp.float32),
            jax.ShapeDtypeStruct((NC, np_), jnp.float32),
            jax.ShapeDtypeStruct((NC, np_), jnp.float32),
        ],
        mesh=mesh,
        compiler_params=pltpu.CompilerParams(needs_layout_passes=False),
        scratch_types=[
            pltpu.VMEM((np_,), jnp.float32),
            pltpu.VMEM((CHUNK,), jnp.int32),
            pltpu.VMEM((CROWS, 128), jnp.int32),
            pltpu.VMEM((CHUNK,), jnp.int32),
            pltpu.VMEM((CROWS, 128), jnp.float32),
            pltpu.VMEM((CHUNK,), jnp.float32),
            pltpu.VMEM((CHUNK,), jnp.float32),
            pltpu.VMEM((128,), jnp.float32),
            pltpu.VMEM_SHARED((np_,), jnp.float32),
            pltpu.VMEM_SHARED((np_,), jnp.float32),
            pltpu.SemaphoreType.DMA,
            pltpu.SemaphoreType.DMA,
        ],
    )
    def k(x_hbm, src1_hbm, dst_hbm, dst1_hbm, ea_hbm, z_hbm,
          s_hbm, d_hbm, deg_hbm, attr_hbm,
          x_v, srcb1, dstb, dstb1, eab, sb, db, ones_v, deg_sh, attr_sh,
          gsem, ssem):
        cid = lax.axis_index("c")
        sid = lax.axis_index("s")
        wid = cid * NS + sid

        @pl.when(sid == 0)
        def _():
            pltpu.sync_copy(z_hbm, deg_sh)
            pltpu.sync_copy(z_hbm, attr_sh)

        pltpu.sync_copy(x_hbm, x_v)
        for i in range(8):
            ones_v[pl.ds(i * 16, 16)] = jnp.ones((16,), jnp.float32)
        plsc.subcore_barrier()

        base = wid * ept
        base_row = wid * erows_pt

        def chunk_body(kk, _):
            off = base + kk * CHUNK
            row = base_row + kk * CROWS
            pltpu.sync_copy(src1_hbm.at[pl.ds(off, CHUNK)], srcb1)
            pltpu.sync_copy(dst_hbm.at[pl.ds(row, CROWS)], dstb)
            pltpu.sync_copy(dst1_hbm.at[pl.ds(off, CHUNK)], dstb1)
            pltpu.sync_copy(ea_hbm.at[pl.ds(row, CROWS)], eab)

            sh = []
            for j in range(CROWS):
                sh.append(pltpu.async_copy(
                    ones_v, deg_sh.at[dstb.at[j]], ssem, add=True))
                sh.append(pltpu.async_copy(
                    eab.at[j], attr_sh.at[dstb.at[j]], ssem, add=True))

            def gat(i, _):
                sidx = srcb1[pl.ds(i * 16, 16)]
                sb[pl.ds(i * 16, 16)] = plsc.load_gather(x_v, [sidx])
                didx = dstb1[pl.ds(i * 16, 16)]
                db[pl.ds(i * 16, 16)] = plsc.load_gather(x_v, [didx])
                return 0

            lax.fori_loop(0, CHUNK // 16, gat, 0)
            pltpu.sync_copy(sb, s_hbm.at[pl.ds(off, CHUNK)])
            pltpu.sync_copy(db, d_hbm.at[pl.ds(off, CHUNK)])
            for h in sh:
                h.wait()
            return 0

        lax.fori_loop(0, n_chunks, chunk_body, 0)
        plsc.subcore_barrier()

        @pl.when(sid == 0)
        def _():
            pltpu.sync_copy(deg_sh, deg_hbm.at[cid])
            pltpu.sync_copy(attr_sh, attr_hbm.at[cid])

    return k(x_p, src1, dst2, dst1, ea2, zeros_na)


def _sc_pass2(dst2, p2, w2, zeros_na, *, np_, ept):
    n_chunks = ept // CHUNK
    erows_pt = ept // 128
    mesh = plsc.VectorSubcoreMesh(core_axis_name="c", subcore_axis_name="s")

    @functools.partial(
        pl.kernel,
        out_type=[
            jax.ShapeDtypeStruct((NC, np_), jnp.float32),
            jax.ShapeDtypeStruct((NC, np_), jnp.float32),
        ],
        mesh=mesh,
        compiler_params=pltpu.CompilerParams(needs_layout_passes=False),
        scratch_types=[
            pltpu.VMEM((CROWS, 128), jnp.int32),
            pltpu.VMEM((CROWS, 128), jnp.float32),
            pltpu.VMEM((CROWS, 128), jnp.float32),
            pltpu.VMEM_SHARED((np_,), jnp.float32),
            pltpu.VMEM_SHARED((np_,), jnp.float32),
            pltpu.SemaphoreType.DMA,
        ],
    )
    def k(dst2_hbm, p2_hbm, w2_hbm, z_hbm, pp_hbm, ss_hbm,
          dstb, pb, wb, p_sh, s_sh, ssem):
        cid = lax.axis_index("c")
        sid = lax.axis_index("s")
        wid = cid * NS + sid

        @pl.when(sid == 0)
        def _():
            pltpu.sync_copy(z_hbm, p_sh)
            pltpu.sync_copy(z_hbm, s_sh)

        plsc.subcore_barrier()
        base_row = wid * erows_pt

        def chunk_body(kk, _):
            row = base_row + kk * CROWS
            pltpu.sync_copy(dst2_hbm.at[pl.ds(row, CROWS)], dstb)
            pltpu.sync_copy(p2_hbm.at[pl.ds(row, CROWS)], pb)
            pltpu.sync_copy(w2_hbm.at[pl.ds(row, CROWS)], wb)

            sh = []
            for j in range(CROWS):
                sh.append(pltpu.async_copy(
                    pb.at[j], p_sh.at[dstb.at[j]], ssem, add=True))
                sh.append(pltpu.async_copy(
                    wb.at[j], s_sh.at[dstb.at[j]], ssem, add=True))
            for h in sh:
                h.wait()
            return 0

        lax.fori_loop(0, n_chunks, chunk_body, 0)
        plsc.subcore_barrier()

        @pl.when(sid == 0)
        def _():
            pltpu.sync_copy(p_sh, pp_hbm.at[cid])
            pltpu.sync_copy(s_sh, ss_hbm.at[cid])

    return k(dst2, p2, w2, zeros_na)


def _raw_block(s, d, a, wl, wr, we, at, bs):
    acc = jnp.zeros_like(s)
    for c in range(32):
        m = s * wl[0, c] + d * wr[0, c] + a * we[0, c] + bs[0, c]
        m = jnp.where(m >= 0.0, m, 0.2 * m)
        acc = acc + m * at[0, c]
    return acc


def _alpha_edges_kernel(wl, wr, we, at, bs, s, d, a, raw, bmax):
    acc = _raw_block(s[...], d[...], a[...], wl, wr, we, at, bs)
    raw[...] = acc
    bmax[...] = jnp.max(acc, axis=0, keepdims=True)


def _alpha_self_kernel(wl, wr, we, at, bs, x, degp, attrp, raw, bmax):
    deg = degp[0] + degp[1]
    asum = attrp[0] + attrp[1]
    la = asum / jnp.maximum(deg, 1.0)
    xv = x[...]
    acc = _raw_block(xv, xv, la, wl, wr, we, at, bs)
    raw[...] = acc
    bmax[...] = jnp.max(acc, axis=0, keepdims=True)


def _exp_kernel(g, raw, s, p, w):
    pv = jnp.exp(raw[...] - g[0, 0])
    p[...] = pv
    w[...] = pv * s[...]


def _final_kernel(g, wl, bl, bias, x, pp, ss, raws, out):
    p_tot = pp[0] + pp[1]
    s_tot = ss[0] + ss[1]
    ps = jnp.exp(raws[...] - g[0, 0])
    p_tot = p_tot + ps
    s_tot = s_tot + ps * x[...]
    denom = p_tot + 1e-16
    sp = s_tot / denom
    pq = p_tot / denom
    for c in range(32):
        out[c] = sp * wl[0, c] + pq * bl[0, c] + bias[0, c]


def kernel(x, edge_index, edge_attr, W_l, b_l, W_r, b_r, W_e, att, bias):
    n = x.shape[0]
    e = edge_index.shape[1]
    np_ = _cdiv(n, 1024) * 1024
    ept = _cdiv(e, NW * CHUNK) * CHUNK
    e_pad = NW * ept
    erows = e_pad // 128
    nrows = np_ // 128

    xf = x[:, 0]
    x_p = jnp.pad(xf, (0, np_ - n))
    pad = e_pad - e
    src = jnp.concatenate([edge_index[0], jnp.zeros((pad,), jnp.int32)])
    pad_dst = (n + (jnp.arange(pad, dtype=jnp.int32) % 256)).astype(jnp.int32)
    dst = jnp.concatenate([edge_index[1], pad_dst])
    ea = jnp.concatenate([edge_attr[:, 0], jnp.zeros((pad,), jnp.float32)])
    dst2 = dst.reshape(erows, 128)
    ea2 = ea.reshape(erows, 128)
    zeros_na = jnp.zeros((np_,), jnp.float32)

    s_arr, d_arr, degp, attrp = _sc_pass1(
        x_p, src, dst2, dst, ea2, zeros_na, np_=np_, ept=ept)
    s2 = s_arr.reshape(erows, 128)
    d2 = d_arr.reshape(erows, 128)

    wl = W_l.reshape(1, 32)
    wr = W_r.reshape(1, 32)
    we = W_e.reshape(1, 32)
    at2 = att.reshape(1, 32)
    bs = (b_l + b_r).reshape(1, 32)
    bl2 = b_l.reshape(1, 32)
    bias2 = bias.reshape(1, 32)

    smem = pl.BlockSpec(memory_space=pltpu.SMEM)
    be = 128
    grid_e = erows // be
    raw2, bmax_e = pl.pallas_call(
        _alpha_edges_kernel,
        grid=(grid_e,),
        in_specs=[smem] * 5 + [
            pl.BlockSpec((be, 128), lambda i: (i, 0)),
            pl.BlockSpec((be, 128), lambda i: (i, 0)),
            pl.BlockSpec((be, 128), lambda i: (i, 0)),
        ],
        out_specs=[
            pl.BlockSpec((be, 128), lambda i: (i, 0)),
            pl.BlockSpec((1, 128), lambda i: (0, i)),
        ],
        out_shape=[
            jax.ShapeDtypeStruct((erows, 128), jnp.float32),
            jax.ShapeDtypeStruct((1, grid_e * 128), jnp.float32),
        ],
        compiler_params=pltpu.CompilerParams(
            dimension_semantics=("parallel",)),
    )(wl, wr, we, at2, bs, s2, d2, ea2)

    x3 = x_p.reshape(nrows, 128)
    degp3 = degp.reshape(NC, nrows, 128)
    attrp3 = attrp.reshape(NC, nrows, 128)
    bn = 112 if nrows % 112 == 0 else nrows
    grid_s = nrows // bn
    raw_self, bmax_s = pl.pallas_call(
        _alpha_self_kernel,
        grid=(grid_s,),
        in_specs=[smem] * 5 + [
            pl.BlockSpec((bn, 128), lambda i: (i, 0)),
            pl.BlockSpec((NC, bn, 128), lambda i: (0, i, 0)),
            pl.BlockSpec((NC, bn, 128), lambda i: (0, i, 0)),
        ],
        out_specs=[
            pl.BlockSpec((bn, 128), lambda i: (i, 0)),
            pl.BlockSpec((1, 128), lambda i: (0, i)),
        ],
        out_shape=[
            jax.ShapeDtypeStruct((nrows, 128), jnp.float32),
            jax.ShapeDtypeStruct((1, grid_s * 128), jnp.float32),
        ],
        compiler_params=pltpu.CompilerParams(
            dimension_semantics=("parallel",)),
    )(wl, wr, we, at2, bs, x3, degp3, attrp3)

    gmax = jnp.maximum(jnp.max(bmax_e), jnp.max(bmax_s)).reshape(1, 1)

    p2, w2 = pl.pallas_call(
        _exp_kernel,
        grid=(grid_e,),
        in_specs=[smem] + [
            pl.BlockSpec((be, 128), lambda i: (i, 0)),
            pl.BlockSpec((be, 128), lambda i: (i, 0)),
        ],
        out_specs=[
            pl.BlockSpec((be, 128), lambda i: (i, 0)),
            pl.BlockSpec((be, 128), lambda i: (i, 0)),
        ],
        out_shape=[
            jax.ShapeDtypeStruct((erows, 128), jnp.float32),
            jax.ShapeDtypeStruct((erows, 128), jnp.float32),
        ],
        compiler_params=pltpu.CompilerParams(
            dimension_semantics=("parallel",)),
    )(gmax, raw2, s2)

    ppart, spart = _sc_pass2(dst2, p2, w2, zeros_na, np_=np_, ept=ept)
    ppart3 = ppart.reshape(NC, nrows, 128)
    spart3 = spart.reshape(NC, nrows, 128)

    bf = 56
    grid_f = nrows // bf if nrows % bf == 0 else 1
    bf = bf if nrows % bf == 0 else nrows
    ot = pl.pallas_call(
        _final_kernel,
        grid=(grid_f,),
        in_specs=[
            smem,
            pl.BlockSpec((1, 32), lambda i: (0, 0), memory_space=pltpu.SMEM),
            pl.BlockSpec((1, 32), lambda i: (0, 0), memory_space=pltpu.SMEM),
            pl.BlockSpec((1, 32), lambda i: (0, 0), memory_space=pltpu.SMEM),
            pl.BlockSpec((bf, 128), lambda i: (i, 0)),
            pl.BlockSpec((NC, bf, 128), lambda i: (0, i, 0)),
            pl.BlockSpec((NC, bf, 128), lambda i: (0, i, 0)),
            pl.BlockSpec((bf, 128), lambda i: (i, 0)),
        ],
        out_specs=pl.BlockSpec((32, bf, 128), lambda i: (0, i, 0)),
        out_shape=jax.ShapeDtypeStruct((32, nrows, 128), jnp.float32),
        compiler_params=pltpu.CompilerParams(
            dimension_semantics=("parallel",)),
    )(gmax, wl, bl2, bias2, x3, ppart3, spart3, raw_self)

    out = ot.transpose(1, 2, 0).reshape(np_, 32)[:n]
    return out.reshape(1, n * 32)


# speedup vs baseline: 153.0836x; 1.6100x over previous
import functools

import jax
import jax.numpy as jnp
from jax import lax
from jax.experimental import pallas as pl
from jax.experimental.pallas import tpu as pltpu
from jax.experimental.pallas import tpu_sc as plsc

NC = 2
NS = 16
NW = NC * NS

CHUNK = 2048
CROWS = CHUNK // 128


def _cdiv(a, b):
    return (a + b - 1) // b


def _sc_gather(x_p, src1, dst1, *, np_, erows):
    n_full = erows // CROWS
    tail = erows % CROWS
    mesh = plsc.VectorSubcoreMesh(core_axis_name="c", subcore_axis_name="s")

    @functools.partial(
        pl.kernel,
        out_type=[
            jax.ShapeDtypeStruct((erows * 128,), jnp.float32),
            jax.ShapeDtypeStruct((erows * 128,), jnp.float32),
        ],
        mesh=mesh,
        compiler_params=pltpu.CompilerParams(needs_layout_passes=False),
        scratch_types=[
            pltpu.VMEM((np_,), jnp.float32),
            pltpu.VMEM((CHUNK,), jnp.int32),
            pltpu.VMEM((CHUNK,), jnp.int32),
            pltpu.VMEM((CHUNK,), jnp.float32),
            pltpu.VMEM((CHUNK,), jnp.float32),
            pltpu.VMEM_SHARED((np_,), jnp.float32),
            pltpu.SemaphoreType.DMA,
        ],
    )
    def k(x_hbm, src1_hbm, dst1_hbm, s_hbm, d_hbm,
          x_v, srcb1, dstb1, sb, db, x_sh, lsem):
        cid = lax.axis_index("c")
        sid = lax.axis_index("s")
        wid = cid * NS + sid

        @pl.when(sid == 0)
        def _():
            pltpu.sync_copy(x_hbm, x_sh)

        plsc.subcore_barrier()
        pltpu.sync_copy(x_sh, x_v)

        def do_rows(off, nrows128):
            cnt = nrows128 * 128
            h1 = pltpu.async_copy(src1_hbm.at[pl.ds(off, cnt)],
                                  srcb1.at[pl.ds(0, cnt)], lsem)
            h2 = pltpu.async_copy(dst1_hbm.at[pl.ds(off, cnt)],
                                  dstb1.at[pl.ds(0, cnt)], lsem)
            h1.wait()
            h2.wait()

            def gat(i, _):
                sidx = srcb1[pl.ds(i * 16, 16)]
                sb[pl.ds(i * 16, 16)] = plsc.load_gather(x_v, [sidx])
                didx = dstb1[pl.ds(i * 16, 16)]
                db[pl.ds(i * 16, 16)] = plsc.load_gather(x_v, [didx])
                return 0

            lax.fori_loop(0, cnt // 16, gat, 0)
            pltpu.sync_copy(sb.at[pl.ds(0, cnt)], s_hbm.at[pl.ds(off, cnt)])
            pltpu.sync_copy(db.at[pl.ds(0, cnt)], d_hbm.at[pl.ds(off, cnt)])

        n_chunks_w = (n_full + NW - 1 - wid) // NW

        def chunk_body(kk, _):
            do_rows((kk * NW + wid) * CHUNK, CROWS)
            return 0

        lax.fori_loop(0, n_chunks_w, chunk_body, 0)
        if tail:
            @pl.when(wid < tail)
            def _():
                do_rows((n_full * CROWS + wid) * 128, 1)

    return k(x_p, src1, dst1)


def _sc_scatter_pair(dst2, v1_2, v2_2, zeros_na, dep, *, np_, erows,
                     deg_mode):
    SROWS = 32
    n_full = erows // SROWS
    tail = erows % SROWS
    mesh = plsc.VectorSubcoreMesh(core_axis_name="c", subcore_axis_name="s")

    @functools.partial(
        pl.kernel,
        out_type=[
            jax.ShapeDtypeStruct((NC, np_), jnp.float32),
            jax.ShapeDtypeStruct((NC, np_), jnp.float32),
        ],
        mesh=mesh,
        compiler_params=pltpu.CompilerParams(needs_layout_passes=False),
        scratch_types=[
            pltpu.VMEM((SROWS, 128), jnp.int32),
            pltpu.VMEM((SROWS, 128), jnp.float32),
            pltpu.VMEM((SROWS, 128), jnp.float32),
            pltpu.VMEM((128,), jnp.float32),
            pltpu.VMEM_SHARED((np_,), jnp.float32),
            pltpu.VMEM_SHARED((np_,), jnp.float32),
            pltpu.SemaphoreType.DMA,
            pltpu.SemaphoreType.DMA,
        ],
    )
    def k(dst2_hbm, v1_hbm, v2_hbm, z_hbm, dep_hbm, a1_hbm, a2_hbm,
          dstb, vb1, vb2, ones_v, sh1, sh2, ssem, lsem):
        cid = lax.axis_index("c")
        sid = lax.axis_index("s")
        wid = cid * NS + sid

        @pl.when(sid == 0)
        def _():
            pltpu.sync_copy(z_hbm, sh1)
            pltpu.sync_copy(z_hbm, sh2)

        if deg_mode:
            for i in range(8):
                ones_v[pl.ds(i * 16, 16)] = jnp.ones((16,), jnp.float32)
        plsc.subcore_barrier()

        def do_rows(row, nr):
            lh = [pltpu.async_copy(dst2_hbm.at[pl.ds(row, nr)],
                                   dstb.at[pl.ds(0, nr)], lsem)]
            if not deg_mode:
                lh.append(pltpu.async_copy(v1_hbm.at[pl.ds(row, nr)],
                                           vb1.at[pl.ds(0, nr)], lsem))
            lh.append(pltpu.async_copy(v2_hbm.at[pl.ds(row, nr)],
                                       vb2.at[pl.ds(0, nr)], lsem))
            for h in lh:
                h.wait()
            sh = []
            for j in range(nr):
                src1_j = ones_v if deg_mode else vb1.at[j]
                sh.append(pltpu.async_copy(
                    src1_j, sh1.at[dstb.at[j]], ssem, add=True))
                sh.append(pltpu.async_copy(
                    vb2.at[j], sh2.at[dstb.at[j]], ssem, add=True))
            for h in sh:
                h.wait()

        n_chunks_w = (n_full + NW - 1 - wid) // NW

        def chunk_body(kk, _):
            do_rows((kk * NW + wid) * SROWS, SROWS)
            return 0

        lax.fori_loop(0, n_chunks_w, chunk_body, 0)
        if tail:
            @pl.when(wid < tail)
            def _():
                do_rows(n_full * SROWS + wid, 1)

        plsc.subcore_barrier()

        @pl.when(sid == 0)
        def _():
            pltpu.sync_copy(sh1, a1_hbm.at[cid])
            pltpu.sync_copy(sh2, a2_hbm.at[cid])

    return k(dst2, v1_2, v2_2, zeros_na, dep)


def _raw_block(s, d, a, wl, wr, we, at, bs):
    cs = cd = ca = cb = 0.0
    for c in range(32):
        cs = cs + at[0, c] * wl[0, c]
        cd = cd + at[0, c] * wr[0, c]
        ca = ca + at[0, c] * we[0, c]
        cb = cb + at[0, c] * bs[0, c]
    lin = s * cs + d * cd + a * ca + cb
    acc = jnp.zeros_like(s)
    for c in range(32):
        m = s * wl[0, c] + d * wr[0, c] + a * we[0, c] + bs[0, c]
        acc = acc + jnp.abs(m) * at[0, c]
    return 0.6 * lin + 0.4 * acc


def _alpha_edges_kernel(wl, wr, we, at, bs, s, d, a, raw, bmax):
    acc = _raw_block(s[...], d[...], a[...], wl, wr, we, at, bs)
    raw[...] = acc
    bmax[...] = jnp.max(acc, axis=0, keepdims=True)


def _alpha_self_kernel(wl, wr, we, at, bs, x, degp, attrp, raw, bmax):
    deg = degp[0] + degp[1]
    asum = attrp[0] + attrp[1]
    la = asum / jnp.maximum(deg, 1.0)
    xv = x[...]
    acc = _raw_block(xv, xv, la, wl, wr, we, at, bs)
    raw[...] = acc
    bmax[...] = jnp.max(acc, axis=0, keepdims=True)


def _exp_kernel(g, raw, s, p, w):
    pv = jnp.exp(raw[...] - g[0, 0])
    p[...] = pv
    w[...] = pv * s[...]


def _final_kernel(g, wl, bl, bias, x, pp, ss, raws, out):
    p_tot = pp[0] + pp[1]
    s_tot = ss[0] + ss[1]
    ps = jnp.exp(raws[...] - g[0, 0])
    p_tot = p_tot + ps
    s_tot = s_tot + ps * x[...]
    denom = p_tot + 1e-16
    sp = s_tot / denom
    pq = p_tot / denom
    for c in range(32):
        out[c] = sp * wl[0, c] + pq * bl[0, c] + bias[0, c]


def kernel(x, edge_index, edge_attr, W_l, b_l, W_r, b_r, W_e, att, bias):
    n = x.shape[0]
    e = edge_index.shape[1]
    np_ = _cdiv(n, 1024) * 1024
    e128 = _cdiv(e, 128) * 128
    erows = e128 // 128
    nrows = np_ // 128

    xf = x[:, 0]
    x_p = jnp.pad(xf, (0, np_ - n))
    if e128 == e:
        src = edge_index[0]
        dst = edge_index[1]
        ea = edge_attr[:, 0]
    else:
        pad = e128 - e
        src = jnp.concatenate([edge_index[0], jnp.zeros((pad,), jnp.int32)])
        pad_dst = (n + (jnp.arange(pad, dtype=jnp.int32) % 256)
                   ).astype(jnp.int32)
        dst = jnp.concatenate([edge_index[1], pad_dst])
        ea = jnp.concatenate([edge_attr[:, 0], jnp.zeros((pad,),
                                                         jnp.float32)])
    dst2 = dst.reshape(erows, 128)
    ea2 = ea.reshape(erows, 128)
    zeros_na = jnp.zeros((np_,), jnp.float32)

    s_arr, d_arr = _sc_gather(x_p, src, dst, np_=np_, erows=erows)
    degp, attrp = _sc_scatter_pair(
        dst2, ea2, ea2, zeros_na, s_arr, np_=np_, erows=erows, deg_mode=True)
    s2 = s_arr.reshape(erows, 128)
    d2 = d_arr.reshape(erows, 128)

    wl = W_l.reshape(1, 32)
    wr = W_r.reshape(1, 32)
    we = W_e.reshape(1, 32)
    at2 = att.reshape(1, 32)
    bs = (b_l + b_r).reshape(1, 32)
    bl2 = b_l.reshape(1, 32)
    bias2 = bias.reshape(1, 32)

    smem = pl.BlockSpec(memory_space=pltpu.SMEM)
    be = None
    for c in range(min(erows, 1000), 7, -1):
        if erows % c == 0 and c % 8 == 0:
            be = c
            break
    if be is None:
        erows_tc = _cdiv(erows, 1000) * 1000
        prows = erows_tc - erows
        s2 = jnp.pad(s2, ((0, prows), (0, 0)))
        d2 = jnp.pad(d2, ((0, prows), (0, 0)))
        ea2_tc = jnp.pad(ea2, ((0, prows), (0, 0)))
        be = 1000
    else:
        erows_tc = erows
        ea2_tc = ea2
    grid_e = erows_tc // be
    raw2, bmax_e = pl.pallas_call(
        _alpha_edges_kernel,
        grid=(grid_e,),
        in_specs=[smem] * 5 + [
            pl.BlockSpec((be, 128), lambda i: (i, 0)),
            pl.BlockSpec((be, 128), lambda i: (i, 0)),
            pl.BlockSpec((be, 128), lambda i: (i, 0)),
        ],
        out_specs=[
            pl.BlockSpec((be, 128), lambda i: (i, 0)),
            pl.BlockSpec((1, 128), lambda i: (0, i)),
        ],
        out_shape=[
            jax.ShapeDtypeStruct((erows_tc, 128), jnp.float32),
            jax.ShapeDtypeStruct((1, grid_e * 128), jnp.float32),
        ],
        compiler_params=pltpu.CompilerParams(
            dimension_semantics=("parallel",)),
    )(wl, wr, we, at2, bs, s2, d2, ea2_tc)

    x3 = x_p.reshape(nrows, 128)
    degp3 = degp.reshape(NC, nrows, 128)
    attrp3 = attrp.reshape(NC, nrows, 128)
    bn = 112 if nrows % 112 == 0 else nrows
    grid_s = nrows // bn
    raw_self, bmax_s = pl.pallas_call(
        _alpha_self_kernel,
        grid=(grid_s,),
        in_specs=[smem] * 5 + [
            pl.BlockSpec((bn, 128), lambda i: (i, 0)),
            pl.BlockSpec((NC, bn, 128), lambda i: (0, i, 0)),
            pl.BlockSpec((NC, bn, 128), lambda i: (0, i, 0)),
        ],
        out_specs=[
            pl.BlockSpec((bn, 128), lambda i: (i, 0)),
            pl.BlockSpec((1, 128), lambda i: (0, i)),
        ],
        out_shape=[
            jax.ShapeDtypeStruct((nrows, 128), jnp.float32),
            jax.ShapeDtypeStruct((1, grid_s * 128), jnp.float32),
        ],
        compiler_params=pltpu.CompilerParams(
            dimension_semantics=("parallel",)),
    )(wl, wr, we, at2, bs, x3, degp3, attrp3)

    gmax = jnp.maximum(jnp.max(bmax_e), jnp.max(bmax_s)).reshape(1, 1)

    p2, w2 = pl.pallas_call(
        _exp_kernel,
        grid=(grid_e,),
        in_specs=[smem] + [
            pl.BlockSpec((be, 128), lambda i: (i, 0)),
            pl.BlockSpec((be, 128), lambda i: (i, 0)),
        ],
        out_specs=[
            pl.BlockSpec((be, 128), lambda i: (i, 0)),
            pl.BlockSpec((be, 128), lambda i: (i, 0)),
        ],
        out_shape=[
            jax.ShapeDtypeStruct((erows_tc, 128), jnp.float32),
            jax.ShapeDtypeStruct((erows_tc, 128), jnp.float32),
        ],
        compiler_params=pltpu.CompilerParams(
            dimension_semantics=("parallel",)),
    )(gmax, raw2, s2)

    ppart, spart = _sc_scatter_pair(
        dst2, p2, w2, zeros_na, zeros_na, np_=np_, erows=erows,
        deg_mode=False)
    ppart3 = ppart.reshape(NC, nrows, 128)
    spart3 = spart.reshape(NC, nrows, 128)

    bf = 56
    grid_f = nrows // bf if nrows % bf == 0 else 1
    bf = bf if nrows % bf == 0 else nrows
    ot = pl.pallas_call(
        _final_kernel,
        grid=(grid_f,),
        in_specs=[
            smem,
            pl.BlockSpec((1, 32), lambda i: (0, 0), memory_space=pltpu.SMEM),
            pl.BlockSpec((1, 32), lambda i: (0, 0), memory_space=pltpu.SMEM),
            pl.BlockSpec((1, 32), lambda i: (0, 0), memory_space=pltpu.SMEM),
            pl.BlockSpec((bf, 128), lambda i: (i, 0)),
            pl.BlockSpec((NC, bf, 128), lambda i: (0, i, 0)),
            pl.BlockSpec((NC, bf, 128), lambda i: (0, i, 0)),
            pl.BlockSpec((bf, 128), lambda i: (i, 0)),
        ],
        out_specs=pl.BlockSpec((32, bf, 128), lambda i: (0, i, 0)),
        out_shape=jax.ShapeDtypeStruct((32, nrows, 128), jnp.float32),
        compiler_params=pltpu.CompilerParams(
            dimension_semantics=("parallel",)),
    )(gmax, wl, bl2, bias2, x3, ppart3, spart3, raw_self)

    out = ot.transpose(1, 2, 0).reshape(np_, 32)[:n]
    return out.reshape(1, n * 32)
